# baseline (speedup 1.0000x reference)
import functools

import jax
import jax.numpy as jnp
from jax import lax
from jax.experimental import pallas as pl
from jax.experimental.pallas import tpu as pltpu
from jax.experimental.pallas import tpu_sc as plsc

D_ROWS = 64
SIZE = 1_000_000
NC, NS, L = 2, 16, 16
NW = NC * NS
W = 512
NWIN = 61
TSPAN = W * NWIN
TAIL0 = NW * TSPAN
TAILW = SIZE - TAIL0

_mesh = plsc.VectorSubcoreMesh(core_axis_name="c", subcore_axis_name="s")


@functools.partial(
    pl.kernel,
    out_type=jax.ShapeDtypeStruct((SIZE,), jnp.float32),
    mesh=_mesh,
    scratch_types=[
        pltpu.VMEM((D_ROWS, W), jnp.float32),
        pltpu.VMEM((D_ROWS, W), jnp.float32),
        pltpu.VMEM((W,), jnp.int32),
        pltpu.VMEM((W,), jnp.int32),
        pltpu.VMEM((W,), jnp.float32),
        pltpu.VMEM((W,), jnp.float32),
        pltpu.VMEM((W,), jnp.float32),
        pltpu.VMEM((W,), jnp.float32),
        pltpu.VMEM((D_ROWS, TAILW), jnp.float32),
        pltpu.VMEM((TAILW,), jnp.int32),
        pltpu.VMEM((TAILW,), jnp.float32),
        pltpu.VMEM((TAILW,), jnp.float32),
        pltpu.VMEM((L,), jnp.int32),
        pltpu.SemaphoreType.DMA,
        pltpu.SemaphoreType.DMA,
    ],
)
def _delay_gather(buf_hbm, delays_hbm, spikes_hbm, ptr_hbm, out_hbm,
                  blkA, blkB, dvA, dvB, svA, svB, ovA, ovB,
                  tblk, tdv, tsv, tov, pv, sem_in, sem_out):
    blk = (blkA, blkB)
    dv = (dvA, dvB)
    sv = (svA, svB)
    ov = (ovA, ovB)
    wid = lax.axis_index("s") * NC + lax.axis_index("c")
    tbase = wid * TSPAN
    pltpu.sync_copy(ptr_hbm, pv)
    ptr_v = pv[...]
    p1 = ptr_v + 1
    ptr_mod = ptr_v & (D_ROWS - 1)

    def cbase_of(k):
        return pl.multiple_of(tbase + k * W, W)

    def fire_in(k, b):
        cbase = cbase_of(k)
        pltpu.async_copy(buf_hbm.at[:, pl.ds(cbase, W)], blk[b], sem_in)
        pltpu.async_copy(delays_hbm.at[pl.ds(cbase, W)], dv[b], sem_in)
        pltpu.async_copy(spikes_hbm.at[pl.ds(cbase, W)], sv[b], sem_in)

    def wait_in(k, b):
        cbase = cbase_of(k)
        pltpu.make_async_copy(buf_hbm.at[:, pl.ds(cbase, W)], blk[b],
                              sem_in).wait()
        pltpu.make_async_copy(delays_hbm.at[pl.ds(cbase, W)], dv[b],
                              sem_in).wait()
        pltpu.make_async_copy(spikes_hbm.at[pl.ds(cbase, W)], sv[b],
                              sem_in).wait()

    def select_tree(blk_ref, d, off):
        r = (p1 - d) & (D_ROWS - 1)
        vals = [blk_ref[row, pl.ds(off, L)] for row in range(D_ROWS)]
        for bit in range(6):
            take_hi = ((r >> bit) & 1) == 1
            vals = [jnp.where(take_hi, vals[2 * i + 1], vals[2 * i])
                    for i in range(len(vals) // 2)]
        return r, vals[0]

    def compute(k, b):
        def group(u, carry):
            off = u * L
            d = dv[b][pl.ds(off, L)]
            r, val = select_tree(blk[b], d, off)
            ov[b][pl.ds(off, L)] = jnp.where(
                r == ptr_mod, sv[b][pl.ds(off, L)], val)
            return carry
        lax.fori_loop(0, W // L, group, 0)

    def fire_out(k, b):
        pltpu.async_copy(ov[b], out_hbm.at[pl.ds(cbase_of(k), W)], sem_out)

    def wait_out():
        pltpu.make_async_copy(ov[0], out_hbm.at[pl.ds(tbase, W)],
                              sem_out).wait()

    fire_in(0, 0)

    def body(j, carry):
        k0 = 2 * j
        k1 = k0 + 1
        fire_in(k1, 1)
        wait_in(k0, 0)

        @pl.when(j >= 1)
        def _():
            wait_out()

        compute(k0, 0)
        fire_out(k0, 0)

        @pl.when(k1 + 1 < NWIN)
        def _():
            fire_in(k0 + 2, 0)

        wait_in(k1, 1)

        @pl.when(j >= 1)
        def _():
            wait_out()

        compute(k1, 1)
        fire_out(k1, 1)
        return carry

    lax.fori_loop(0, NWIN // 2, body, 0)
    wait_in(NWIN - 1, 0)
    wait_out()
    compute(NWIN - 1, 0)
    fire_out(NWIN - 1, 0)
    wait_out()
    wait_out()

    @pl.when(wid == NW - 1)
    def _tail():
        pltpu.async_copy(buf_hbm.at[:, pl.ds(TAIL0, TAILW)], tblk, sem_in)
        pltpu.async_copy(delays_hbm.at[pl.ds(TAIL0, TAILW)], tdv, sem_in)
        pltpu.async_copy(spikes_hbm.at[pl.ds(TAIL0, TAILW)], tsv, sem_in)
        pltpu.make_async_copy(buf_hbm.at[:, pl.ds(TAIL0, TAILW)], tblk,
                              sem_in).wait()
        pltpu.make_async_copy(delays_hbm.at[pl.ds(TAIL0, TAILW)], tdv,
                              sem_in).wait()
        pltpu.make_async_copy(spikes_hbm.at[pl.ds(TAIL0, TAILW)], tsv,
                              sem_in).wait()

        def tgroup(u, carry):
            off = u * L
            d = tdv[pl.ds(off, L)]
            r, val = select_tree(tblk, d, off)
            tov[pl.ds(off, L)] = jnp.where(
                r == ptr_mod, tsv[pl.ds(off, L)], val)
            return carry
        lax.fori_loop(0, TAILW // L, tgroup, 0)
        pltpu.sync_copy(tov, out_hbm.at[pl.ds(TAIL0, TAILW)])


def kernel(buffer, spikes, delays, ptr):
    ptr_b = jnp.full((L,), ptr, dtype=jnp.int32)
    return _delay_gather(buffer, delays.astype(jnp.int32),
                         spikes.astype(jnp.float32), ptr_b)
